# baseline (speedup 1.0000x reference)
import functools

import jax
import jax.numpy as jnp
from jax import lax
from jax.experimental import pallas as pl
from jax.experimental.pallas import tpu as pltpu
from jax.experimental.pallas import tpu_sc as plsc

NC = 2
NS = 16
NW = NC * NS
LANES = 16

CHUNK = 32
IDX_MINOR = 112


def _sc_scores(b, k1, d, rows_per_w, nchunk, v2):
  n_gath = (CHUNK * k1) // IDX_MINOR
  d2 = 2 * d

  mesh = plsc.VectorSubcoreMesh(core_axis_name="c", subcore_axis_name="s")

  @functools.partial(
      pl.kernel,
      out_type=jax.ShapeDtypeStruct((b, 2 * LANES), jnp.float32),
      mesh=mesh,
      compiler_params=pltpu.CompilerParams(needs_layout_passes=False),
      scratch_types=[
          pltpu.VMEM((CHUNK + LANES,), jnp.int32),
          pltpu.VMEM((CHUNK * k1,), jnp.int32),
          pltpu.VMEM((CHUNK,), jnp.int32),
          pltpu.VMEM((CHUNK * k1,), jnp.int32),
          pltpu.VMEM((CHUNK, d2), jnp.float32),
          pltpu.VMEM((CHUNK * k1, d2), jnp.float32),
          pltpu.VMEM((CHUNK, 2 * LANES), jnp.float32),
          pltpu.SemaphoreType.DMA,
      ],
  )
  def sc_kernel(tidx_hbm, aidx_hbm, in2_hbm, out2_hbm, scores_hbm,
                tidx_v, aidx_v, th_v, ah_v, t_v, a_v, sc_v, sem):
    wid = lax.axis_index("s") * NC + lax.axis_index("c")
    lane = lax.iota(jnp.int32, LANES)

    def chunk_body(i, _):
      base = wid * rows_per_w + i * CHUNK
      pltpu.sync_copy(tidx_hbm.at[pl.ds(base, CHUNK)],
                      tidx_v.at[pl.ds(0, CHUNK)])
      pltpu.sync_copy(aidx_hbm.at[pl.ds(base * k1, CHUNK * k1)], aidx_v)
      for j in range(CHUNK // LANES):
        tv = tidx_v[pl.ds(j * LANES, LANES)]
        th_v[pl.ds(j * LANES, LANES)] = jnp.where(tv >= v2, tv - v2, tv)
      for j in range((CHUNK * k1) // LANES):
        av = aidx_v[pl.ds(j * LANES, LANES)]
        ah_v[pl.ds(j * LANES, LANES)] = jnp.where(av >= v2, av - v2, av)
      cps = [pltpu.async_copy(in2_hbm.at[th_v], t_v, sem)]
      for j in range(n_gath):
        cps.append(
            pltpu.async_copy(
                out2_hbm.at[ah_v.at[pl.ds(j * IDX_MINOR, IDX_MINOR)]],
                a_v.at[pl.ds(j * IDX_MINOR, IDX_MINOR)],
                sem,
            ))
      for cp in cps:
        cp.wait()

      def row_body(r, _):
        zeros = jnp.zeros((LANES,), jnp.int32)
        dfill = jnp.full((LANES,), d, jnp.int32)
        toffs = jnp.where(tidx_v[pl.ds(r, LANES)] >= v2, dfill, zeros)
        va = jnp.where(aidx_v[pl.ds(r * k1, LANES)] >= v2, dfill, zeros)
        vb = jnp.where(
            aidx_v[pl.ds(r * k1 + (k1 - LANES), LANES)] >= v2, dfill, zeros)
        toff = toffs[0]
        t0 = t_v[r, pl.ds(toff, LANES)]
        t1 = t_v[r, pl.ds(toff + LANES, LANES)]
        t2 = t_v[r, pl.ds(toff + 2 * LANES, LANES)]
        t3 = t_v[r, pl.ds(toff + 3 * LANES, LANES)]
        acc_a = jnp.zeros((LANES,), jnp.float32)
        acc_b = jnp.zeros((LANES,), jnp.float32)
        for k in range(k1):
          row = r * k1 + k
          aoff = va[k] if k < LANES else vb[k - (k1 - LANES)]
          p = t0 * a_v[row, pl.ds(aoff, LANES)]
          p += t1 * a_v[row, pl.ds(aoff + LANES, LANES)]
          p += t2 * a_v[row, pl.ds(aoff + 2 * LANES, LANES)]
          p += t3 * a_v[row, pl.ds(aoff + 3 * LANES, LANES)]
          s = plsc.cumsum(p)[jnp.full((LANES,), LANES - 1, jnp.int32)]
          if k < LANES:
            acc_a = jnp.where(lane == k, s, acc_a)
          else:
            acc_b = jnp.where(lane == (k - LANES), s, acc_b)
        sc_v[r, pl.ds(0, LANES)] = acc_a
        sc_v[r, pl.ds(LANES, LANES)] = acc_b
        return 0

      lax.fori_loop(0, CHUNK, row_body, 0)
      pltpu.sync_copy(sc_v, scores_hbm.at[pl.ds(base, CHUNK)])
      return 0

    lax.fori_loop(0, nchunk, chunk_body, 0)

  return sc_kernel


def _tr_body(a_ref, b_ref, out_ref):
  out_ref[...] = jnp.concatenate([a_ref[...].T, b_ref[...].T], axis=1)


def _pair_transpose(table_t, w):
  d, v = table_t.shape
  grid = (v + 2 * w - 1) // (2 * w)
  v2_blocks = grid
  return pl.pallas_call(
      _tr_body,
      grid=(grid,),
      in_specs=[
          pl.BlockSpec((d, w), lambda g: (0, g)),
          pl.BlockSpec((d, w), lambda g: (0, g + v2_blocks)),
      ],
      out_specs=pl.BlockSpec((w, 2 * d), lambda g: (g, 0)),
      out_shape=jax.ShapeDtypeStruct((grid * w, 2 * d), jnp.float32),
  )(table_t, table_t)


def _loss_body(nk, b, s_ref, o_ref):
  s = s_ref[...]
  col = lax.broadcasted_iota(jnp.int32, s.shape, 1)
  x = jnp.where(col == nk, -s, s)
  sp = jnp.maximum(x, 0.0) + jnp.log1p(jnp.exp(-jnp.abs(x)))
  sp = jnp.where(col <= nk, sp, 0.0)
  o_ref[...] = (jnp.sum(sp) / b).reshape(1, 1)


def kernel(target, context, negative_samples, in_emb, out_emb):
  b, k = negative_samples.shape
  v, d = in_emb.shape
  k1 = k + 1
  rows_per_w = b // NW
  nchunk = rows_per_w // CHUNK

  in2 = _pair_transpose(in_emb.T, 512)
  out2 = _pair_transpose(out_emb.T, 512)
  v2 = in2.shape[0]

  idx_all = jnp.concatenate([negative_samples, context[:, None]], axis=1)
  idx_all = idx_all.reshape(b * k1)

  scores = _sc_scores(b, k1, d, rows_per_w, nchunk, v2)(
      target, idx_all, in2, out2)

  loss = pl.pallas_call(
      functools.partial(_loss_body, k, b),
      out_shape=jax.ShapeDtypeStruct((1, 1), jnp.float32),
  )(scores)
  return loss[0, 0]


# speedup vs baseline: 1.3173x; 1.3173x over previous
import functools

import jax
import jax.numpy as jnp
from jax import lax
from jax.experimental import pallas as pl
from jax.experimental.pallas import tpu as pltpu
from jax.experimental.pallas import tpu_sc as plsc

NC = 2
NS = 16
NW = NC * NS
LANES = 16

CHUNK = 32
IDX_MINOR = 112


def _sc_scores(b, k1, d, rows_per_w, nchunk):
  n_gath = (CHUNK * k1) // IDX_MINOR
  assert nchunk % 2 == 0 and nchunk >= 4

  mesh = plsc.VectorSubcoreMesh(core_axis_name="c", subcore_axis_name="s")

  @functools.partial(
      pl.kernel,
      out_type=jax.ShapeDtypeStruct((b, 2 * LANES), jnp.float32),
      mesh=mesh,
      compiler_params=pltpu.CompilerParams(
          needs_layout_passes=False, use_tc_tiling_on_sc=False),
      scratch_types=[
          pltpu.VMEM((2, CHUNK), jnp.int32),
          pltpu.VMEM((2, CHUNK * k1), jnp.int32),
          pltpu.VMEM((2, CHUNK, d), jnp.float32),
          pltpu.VMEM((2, CHUNK * k1, d), jnp.float32),
          pltpu.VMEM((CHUNK, 2 * LANES), jnp.float32),
          pltpu.SemaphoreType.DMA,
          pltpu.SemaphoreType.DMA,
      ],
  )
  def sc_kernel(tidx_hbm, aidx_hbm, in_emb_hbm, out_emb_hbm, scores_hbm,
                tidx_v, aidx_v, t_v, a_v, sc_v, sem0, sem1):
    wid = lax.axis_index("s") * NC + lax.axis_index("c")
    lane = lax.iota(jnp.int32, LANES)
    sems = (sem0, sem1)

    def issue(i, s):
      base = wid * rows_per_w + i * CHUNK
      sem = sems[s]
      pltpu.sync_copy(tidx_hbm.at[pl.ds(base, CHUNK)], tidx_v.at[s])
      pltpu.sync_copy(aidx_hbm.at[pl.ds(base * k1, CHUNK * k1)],
                      aidx_v.at[s])
      pltpu.async_copy(in_emb_hbm.at[tidx_v.at[s]], t_v.at[s], sem)
      for j in range(n_gath):
        pltpu.async_copy(
            out_emb_hbm.at[aidx_v.at[s].at[pl.ds(j * IDX_MINOR, IDX_MINOR)]],
            a_v.at[s].at[pl.ds(j * IDX_MINOR, IDX_MINOR)],
            sem,
        )

    def wait(s):
      sem = sems[s]
      pltpu.make_async_copy(in_emb_hbm.at[tidx_v.at[s]], t_v.at[s],
                            sem).wait()
      for j in range(n_gath):
        pltpu.make_async_copy(
            out_emb_hbm.at[aidx_v.at[s].at[pl.ds(j * IDX_MINOR, IDX_MINOR)]],
            a_v.at[s].at[pl.ds(j * IDX_MINOR, IDX_MINOR)],
            sem,
        ).wait()

    def compute(i, s):
      base = wid * rows_per_w + i * CHUNK

      def row_body(r, _):
        t0 = t_v[s, r, pl.ds(0, LANES)]
        t1 = t_v[s, r, pl.ds(LANES, LANES)]
        t2 = t_v[s, r, pl.ds(2 * LANES, LANES)]
        t3 = t_v[s, r, pl.ds(3 * LANES, LANES)]
        acc_a = jnp.zeros((LANES,), jnp.float32)
        acc_b = jnp.zeros((LANES,), jnp.float32)
        for k in range(k1):
          row = r * k1 + k
          p = t0 * a_v[s, row, pl.ds(0, LANES)]
          p += t1 * a_v[s, row, pl.ds(LANES, LANES)]
          p += t2 * a_v[s, row, pl.ds(2 * LANES, LANES)]
          p += t3 * a_v[s, row, pl.ds(3 * LANES, LANES)]
          t = plsc.cumsum(p)[jnp.full((LANES,), LANES - 1, jnp.int32)]
          if k < LANES:
            acc_a = jnp.where(lane == k, t, acc_a)
          else:
            acc_b = jnp.where(lane == (k - LANES), t, acc_b)
        sc_v[r, pl.ds(0, LANES)] = acc_a
        sc_v[r, pl.ds(LANES, LANES)] = acc_b
        return 0

      lax.fori_loop(0, CHUNK, row_body, 0)
      pltpu.sync_copy(sc_v, scores_hbm.at[pl.ds(base, CHUNK)])

    issue(0, 0)
    issue(1, 1)

    def pair_body(g, _):
      i = 2 * g
      wait(0)
      issue_i = i + 2
      compute(i, 0)
      issue(issue_i, 0)
      wait(1)
      compute(i + 1, 1)
      issue(issue_i + 1, 1)
      return 0

    lax.fori_loop(0, nchunk // 2 - 1, pair_body, 0)
    wait(0)
    compute(nchunk - 2, 0)
    wait(1)
    compute(nchunk - 1, 1)

  return sc_kernel


def _loss_body(nk, b, s_ref, o_ref):
  s = s_ref[...]
  col = lax.broadcasted_iota(jnp.int32, s.shape, 1)
  x = jnp.where(col == nk, -s, s)
  sp = jnp.maximum(x, 0.0) + jnp.log1p(jnp.exp(-jnp.abs(x)))
  sp = jnp.where(col <= nk, sp, 0.0)
  o_ref[...] = (jnp.sum(sp) / b).reshape(1, 1)


def kernel(target, context, negative_samples, in_emb, out_emb):
  b, k = negative_samples.shape
  d = in_emb.shape[1]
  k1 = k + 1
  rows_per_w = b // NW
  nchunk = rows_per_w // CHUNK

  idx_all = jnp.concatenate([negative_samples, context[:, None]], axis=1)
  idx_all = idx_all.reshape(b * k1)

  scores = _sc_scores(b, k1, d, rows_per_w, nchunk)(
      target, idx_all, in_emb, out_emb)

  loss = pl.pallas_call(
      functools.partial(_loss_body, k, b),
      out_shape=jax.ShapeDtypeStruct((1, 1), jnp.float32),
  )(scores)
  return loss[0, 0]
